# baseline (speedup 1.0000x reference)
import functools

import jax
import jax.numpy as jnp
from jax import lax
from jax.experimental import pallas as pl
from jax.experimental.pallas import tpu as pltpu
from jax.experimental.pallas import tpu_sc as plsc

D = 768
N = 4 * 8192
NC, NS = 2, 16
NW = NC * NS
PER_W = N // NW
C = 32
NCH = PER_W // C


def _sc_fused(idx3d, x2d, table):
    mesh = plsc.VectorSubcoreMesh(core_axis_name="c", subcore_axis_name="s")

    @functools.partial(
        pl.kernel,
        out_type=jax.ShapeDtypeStruct((N, D), jnp.float32),
        mesh=mesh,
        scratch_types=[
            pltpu.VMEM((NCH, C), jnp.int32),
            pltpu.VMEM((2, C, D), jnp.float32),
            pltpu.VMEM((2, C, D), jnp.float32),
            pltpu.SemaphoreType.DMA((2,)),
            pltpu.SemaphoreType.DMA((2,)),
            pltpu.SemaphoreType.DMA((2,)),
        ],
    )
    def fused_kernel(idx_hbm, x_hbm, table_hbm, out_hbm,
                     idx_v, xb, rb, semx, semg, semo):
        wid = lax.axis_index("s") * NC + lax.axis_index("c")
        base = wid * PER_W
        pltpu.sync_copy(idx_hbm.at[wid], idx_v)

        def rows(t):
            return pl.ds(base + t * C, C)

        def start_in(t, p):
            pltpu.async_copy(x_hbm.at[rows(t)], xb.at[p], semx.at[p])
            pltpu.async_copy(table_hbm.at[idx_v.at[t]], rb.at[p], semg.at[p])

        def wait_in(t, p):
            pltpu.make_async_copy(x_hbm.at[rows(t)], xb.at[p], semx.at[p]).wait()
            pltpu.make_async_copy(
                table_hbm.at[idx_v.at[t]], rb.at[p], semg.at[p]).wait()

        def start_out(t, p):
            pltpu.async_copy(xb.at[p], out_hbm.at[rows(t)], semo.at[p])

        def wait_out(t, p):
            pltpu.make_async_copy(xb.at[p], out_hbm.at[rows(t)], semo.at[p]).wait()

        def add(p):
            @pl.loop(0, C)
            def _(r):
                @plsc.parallel_loop(0, D, step=16, unroll=8)
                def _(j):
                    sl = pl.ds(j, 16)
                    plsc.addupdate(xb.at[p, r, sl], rb[p, r, sl])

        start_in(0, 0)

        @pl.loop(0, NCH, step=2)
        def _(t0):
            for p in (0, 1):
                t = t0 + p
                q = 1 - p

                @pl.when(t > 0)
                def _():
                    wait_out(t - 1, q)

                @pl.when(t + 1 < NCH)
                def _():
                    start_in(t + 1, q)

                wait_in(t, p)
                add(p)
                start_out(t, p)

        wait_out(NCH - 1, (NCH - 1) % 2)

    return fused_kernel(idx3d, x2d, table)


def kernel(x, pe_index, pe_weight):
    b, s, d = x.shape
    x2d = x.reshape(N, D)
    idx3d = pe_index.reshape(NW, NCH, C).astype(jnp.int32)
    out = _sc_fused(idx3d, x2d, pe_weight)
    return out.reshape(b, s, d)


# speedup vs baseline: 1.0221x; 1.0221x over previous
import functools

import jax
import jax.numpy as jnp
from jax import lax
from jax.experimental import pallas as pl
from jax.experimental.pallas import tpu as pltpu
from jax.experimental.pallas import tpu_sc as plsc

D = 768
N = 4 * 8192
NC, NS = 2, 16
NW = NC * NS
PER_W = N // NW
C = 16
NCH = PER_W // C
NB = 4
PD = 2


def _sc_fused(idx3d, x2d, table):
    mesh = plsc.VectorSubcoreMesh(core_axis_name="c", subcore_axis_name="s")

    @functools.partial(
        pl.kernel,
        out_type=jax.ShapeDtypeStruct((N, D), jnp.float32),
        mesh=mesh,
        scratch_types=[
            pltpu.VMEM((NCH, C), jnp.int32),
            pltpu.VMEM((NB, C, D), jnp.float32),
            pltpu.VMEM((NB, C, D), jnp.float32),
            pltpu.SemaphoreType.DMA((NB,)),
            pltpu.SemaphoreType.DMA((NB,)),
            pltpu.SemaphoreType.DMA((NB,)),
        ],
    )
    def fused_kernel(idx_hbm, x_hbm, table_hbm, out_hbm,
                     idx_v, xb, rb, semx, semg, semo):
        wid = lax.axis_index("s") * NC + lax.axis_index("c")
        base = wid * PER_W
        pltpu.sync_copy(idx_hbm.at[wid], idx_v)

        def rows(t):
            return pl.ds(base + t * C, C)

        def start_in(t, b):
            pltpu.async_copy(x_hbm.at[rows(t)], xb.at[b], semx.at[b])
            pltpu.async_copy(table_hbm.at[idx_v.at[t]], rb.at[b], semg.at[b])

        def wait_in(t, b):
            pltpu.make_async_copy(x_hbm.at[rows(t)], xb.at[b], semx.at[b]).wait()
            pltpu.make_async_copy(
                table_hbm.at[idx_v.at[t]], rb.at[b], semg.at[b]).wait()

        def start_out(t, b):
            pltpu.async_copy(xb.at[b], out_hbm.at[rows(t)], semo.at[b])

        def wait_out(t, b):
            pltpu.make_async_copy(xb.at[b], out_hbm.at[rows(t)], semo.at[b]).wait()

        def add(b):
            @pl.loop(0, C)
            def _(r):
                @plsc.parallel_loop(0, D, step=16, unroll=8)
                def _(j):
                    sl = pl.ds(j, 16)
                    plsc.addupdate(xb.at[b, r, sl], rb[b, r, sl])

        for t in range(PD):
            start_in(t, t % NB)

        @pl.loop(0, NCH, step=NB)
        def _(t0):
            for k in range(NB):
                t = t0 + k
                b = k
                bp = (k + PD) % NB

                @pl.when(jnp.logical_and(t + PD < NCH, t + PD - NB >= 0))
                def _():
                    wait_out(t + PD - NB, bp)

                @pl.when(t + PD < NCH)
                def _():
                    start_in(t + PD, bp)

                wait_in(t, b)
                add(b)
                start_out(t, b)

        for t in range(NCH - NB, NCH):
            wait_out(t, t % NB)

    return fused_kernel(idx3d, x2d, table)


def kernel(x, pe_index, pe_weight):
    b, s, d = x.shape
    x2d = x.reshape(N, D)
    idx3d = pe_index.reshape(NW, NCH, C).astype(jnp.int32)
    out = _sc_fused(idx3d, x2d, pe_weight)
    return out.reshape(b, s, d)
